# Initial kernel scaffold; baseline (speedup 1.0000x reference)
import jax
import jax.numpy as jnp
from jax.experimental import pallas as pl


def kernel(x, edge_index, edge_weigth, batch, W1, b1, g1, be1, W2, b2, g2, be2, W3, b3, g3, be3, W4, b4, g4, be4, Wf1, bf1, Wf2, bf2, Wf3, bf3):
    raise NotImplementedError("write your pallas kernel here")


# speedup vs baseline: 2.1297x; 2.1297x over previous
import functools

import jax
import jax.numpy as jnp
from jax import lax
from jax.experimental import pallas as pl
from jax.experimental.pallas import tpu as pltpu
from jax.experimental.pallas import tpu_sc as plsc

N = 10000
E = 160000
G = 64

NC = 2
NS = 16
K = 128
E_PAD = 161792
EPT = E_PAD // NS
RPT = 632


@functools.lru_cache(maxsize=None)
def _make_edge_agg(C, F):
    mesh = plsc.VectorSubcoreMesh(core_axis_name="c", subcore_axis_name="s",
                                  num_cores=NC, num_subcores=NS)
    CPC = (C + NC - 1) // NC

    def body(hwc, srcp, dstp, ewp, out, src_v, dst_v, ew_v, rows_v,
             agg_sh, sem):
        core = lax.axis_index("c")
        sid = lax.axis_index("s")

        for cl in range(CPC):
            c = cl * NC + core
            odd_tail = cl == CPC - 1 and C % NC != 0

            def _chunk_pass(c=c):
                @pl.loop(0, K)
                def _zero_rows(r):
                    for j in range(F // 16):
                        rows_v[r, pl.ds(j * 16, 16)] = jnp.zeros(
                            (16,), jnp.float32)

                row0 = sid * RPT
                for zb in range(4):
                    pltpu.sync_copy(rows_v,
                                    agg_sh.at[pl.ds(row0 + zb * K, K)])

                @pl.when(sid < NS - 1)
                def _():
                    pltpu.sync_copy(rows_v.at[pl.ds(0, RPT - 4 * K)],
                                    agg_sh.at[pl.ds(row0 + 4 * K,
                                                    RPT - 4 * K)])

                @pl.when(sid == NS - 1)
                def _():
                    pltpu.sync_copy(rows_v.at[pl.ds(0, 8)],
                                    agg_sh.at[pl.ds(row0 + 4 * K, 8)])

                plsc.subcore_barrier()

                @pl.loop(0, EPT // K)
                def _edge_block(blk):
                    base = sid * EPT + blk * K
                    pltpu.sync_copy(srcp.at[pl.ds(base, K)], src_v)
                    pltpu.sync_copy(dstp.at[pl.ds(base, K)], dst_v)
                    pltpu.sync_copy(ewp.at[pl.ds(base, K)], ew_v)
                    pltpu.async_copy(hwc.at[c].at[src_v], rows_v, sem).wait()

                    @pl.loop(0, K, step=16)
                    def _scale(i0):
                        wv = ew_v[pl.ds(i0, 16)]
                        for e in range(16):
                            w = wv[e]
                            for j in range(F // 16):
                                sl = pl.ds(j * 16, 16)
                                rows_v[i0 + e, sl] = rows_v[i0 + e, sl] * w

                    pltpu.sync_copy(rows_v, agg_sh.at[dst_v], add=True)

                plsc.subcore_barrier()

                @pl.when(sid < NS - 1)
                def _():
                    pltpu.sync_copy(agg_sh.at[pl.ds(row0, RPT)],
                                    out.at[c].at[pl.ds(row0, RPT)])

                @pl.when(sid == NS - 1)
                def _():
                    pltpu.sync_copy(agg_sh.at[pl.ds(row0, 520)],
                                    out.at[c].at[pl.ds(row0, 520)])

            if odd_tail:
                pl.when(c < C)(_chunk_pass)
            else:
                _chunk_pass()

    return pl.kernel(
        body, mesh=mesh,
        out_type=jax.ShapeDtypeStruct((C, N, F), jnp.float32),
        scratch_types=[
            pltpu.VMEM((K,), jnp.int32),
            pltpu.VMEM((K,), jnp.int32),
            pltpu.VMEM((K,), jnp.float32),
            pltpu.VMEM((K, F), jnp.float32),
            pltpu.VMEM_SHARED((N, F), jnp.float32),
            pltpu.SemaphoreType.DMA,
        ])


def _make_matmul(Din, Cout, Fout):
    NB = 5
    Nb = N // NB

    def body(h_ref, w_ref, o_ref):
        o_ref[0] = jnp.dot(h_ref[...].astype(jnp.bfloat16),
                           w_ref[...].astype(jnp.bfloat16),
                           preferred_element_type=jnp.float32)

    return pl.pallas_call(
        body,
        grid=(NB, Cout),
        in_specs=[
            pl.BlockSpec((Nb, Din), lambda nb, co: (nb, 0)),
            pl.BlockSpec((Din, Fout), lambda nb, co: (0, co)),
        ],
        out_specs=pl.BlockSpec((1, Nb, Fout), lambda nb, co: (co, nb, 0)),
        out_shape=jax.ShapeDtypeStruct((Cout, N, Fout), jnp.float32))


def _make_bn_leaky(C, F):
    def body(x_ref, g_ref, be_ref, o_ref):
        xb = x_ref[0]
        mu = jnp.mean(xb, axis=0, keepdims=True)
        xc = xb - mu
        var = jnp.mean(xc * xc, axis=0, keepdims=True)
        y = xc / jnp.sqrt(var + 1e-5) * g_ref[0] + be_ref[0]
        o_ref[...] = jnp.where(y >= 0, y, 0.01 * y)

    return pl.pallas_call(
        body,
        grid=(C,),
        in_specs=[
            pl.BlockSpec((1, N, F), lambda c: (c, 0, 0)),
            pl.BlockSpec((1, 1, F), lambda c: (c, 0, 0)),
            pl.BlockSpec((1, 1, F), lambda c: (c, 0, 0)),
        ],
        out_specs=pl.BlockSpec((N, F), lambda c: (0, c)),
        out_shape=jax.ShapeDtypeStruct((N, C * F), jnp.float32))


def _pool_mlp_body(h_ref, b_ref, w1, b1r, w2, b2r, w3, b3r, o_ref):
    bid = b_ref[...]
    oh = (lax.broadcasted_iota(jnp.int32, (G, N), 0) == bid
          ).astype(jnp.float32)
    y = jnp.dot(oh, h_ref[...], preferred_element_type=jnp.float32,
                precision=lax.Precision.HIGHEST)
    y = jnp.dot(y.astype(jnp.bfloat16), w1[...].astype(jnp.bfloat16),
                preferred_element_type=jnp.float32) + b1r[...]
    y = jnp.where(y >= 0, y, 0.01 * y)
    y = jnp.dot(y.astype(jnp.bfloat16), w2[...].astype(jnp.bfloat16),
                preferred_element_type=jnp.float32) + b2r[...]
    y = jnp.where(y >= 0, y, 0.01 * y)
    y = jnp.dot(y.astype(jnp.bfloat16), w3[...].astype(jnp.bfloat16),
                preferred_element_type=jnp.float32) + b3r[...]
    o_ref[...] = jnp.where(y >= 0, y, 0.01 * y)


_pool_mlp = pl.pallas_call(
    _pool_mlp_body, out_shape=jax.ShapeDtypeStruct((G, 2), jnp.float32))

_L1 = (512, 5, 128)
_L2 = (640, 4, 128)
_L3 = (512, 2, 128)
_L4 = (256, 2, 128)

_mm = [_make_matmul(di, co, fo) for (di, co, fo) in (_L1, _L2, _L3, _L4)]
_bn = [_make_bn_leaky(co, fo) for (_, co, fo) in (_L1, _L2, _L3, _L4)]


def kernel(x, edge_index, edge_weigth, batch,
           W1, b1, g1, be1, W2, b2, g2, be2, W3, b3, g3, be3,
           W4, b4, g4, be4, Wf1, bf1, Wf2, bf2, Wf3, bf3):
    pad = E_PAD - E
    srcp = jnp.concatenate([edge_index[0].astype(jnp.int32),
                            jnp.zeros((pad,), jnp.int32)])
    dstp = jnp.concatenate([edge_index[1].astype(jnp.int32),
                            jnp.zeros((pad,), jnp.int32)])
    ewp = jnp.concatenate([edge_weigth, jnp.zeros((pad,), jnp.float32)])
    batch2 = batch.reshape(1, N).astype(jnp.int32)

    h = x
    Ws = [W1, W2, W3, W4]
    gs = [g1, g2, g3, g4]
    bes = [be1, be2, be3, be4]
    for i, (di, co, fo) in enumerate((_L1, _L2, _L3, _L4)):
        hw = _mm[i](h, Ws[i])
        agg = _make_edge_agg(co, fo)(hw, srcp, dstp, ewp)
        h = _bn[i](agg, gs[i].reshape(co, 1, fo), bes[i].reshape(co, 1, fo))

    return _pool_mlp(h, batch2,
                     Wf1, bf1.reshape(1, 128),
                     Wf2, bf2.reshape(1, 64),
                     Wf3, bf3.reshape(1, 2))
